# baseline (speedup 1.0000x reference)
import jax
import jax.numpy as jnp
from jax import lax
from jax.experimental import pallas as pl
from jax.experimental.pallas import tpu as pltpu
from jax.experimental.pallas import tpu_sc as plsc

_D0, _D1, _D2, _C = 128, 128, 128, 32
_NVOX = _D0 * _D1 * _D2
_HALF = _NVOX // 2
_NC, _NS, _L = 2, 16, 16
_N = 200000
_PBLK = 2048
_G = 128
_GLAST = _N // _PBLK
_SLOTS = _PBLK * _G
_LROW = _SLOTS // 128
_BL = _PBLK // 128
_RSUB = _LROW // _NS
_ZB = 4096
_SPM = _HALF + 16
_ZREP = _HALF // _NS // _ZB


def _lin_body(idx_ref, loc_ref):
    i = pl.program_id(0)
    b = idx_ref[...]
    t = b.T
    lin = t[0:1, :] * (_D1 * _D2) + t[1:2, :] * _D2 + t[2:3, :]
    lin = jnp.reshape(lin, (_BL, 128))
    pid = (
        i * _PBLK
        + lax.broadcasted_iota(jnp.int32, (_BL, 128), 0) * 128
        + lax.broadcasted_iota(jnp.int32, (_BL, 128), 1)
    )
    dump = _HALF + (pid & 15)
    valid = pid < _N
    loc0 = jnp.where(valid & (lin < _HALF), lin, dump)
    loc1 = jnp.where(valid & (lin >= _HALF), lin - _HALF, dump)
    loc_ref[...] = jnp.concatenate([loc0, loc1], axis=0)


_tc_linearize = pl.pallas_call(
    _lin_body,
    grid=(_G,),
    in_specs=[pl.BlockSpec((_PBLK, 3), lambda i: (jnp.minimum(i, _GLAST), 0))],
    out_specs=pl.BlockSpec((2 * _BL, 128), lambda i: (i, 0)),
    out_shape=jax.ShapeDtypeStruct((2 * _LROW, 128), jnp.int32),
    compiler_params=pltpu.CompilerParams(
        dimension_semantics=("arbitrary",),
    ),
)


def _sc_body(loc_hbm, mask_hbm, shared, lin_v, ones_v, zb_v):
    c = lax.axis_index("c")
    s = lax.axis_index("s")

    def _zset(i, carry):
        zb_v[pl.ds(i * _L, _L)] = jnp.zeros((_L,), jnp.float32)
        return carry

    lax.fori_loop(0, _ZB // _L, _zset, 0)
    for t in range(128 // _L):
        ones_v[pl.ds(t * _L, _L)] = jnp.ones((_L,), jnp.float32)
    zbase = s * (_HALF // _NS)
    for r in range(_ZREP):
        pltpu.sync_copy(zb_v, shared.at[pl.ds(zbase + r * _ZB, _ZB)])
    plsc.subcore_barrier()

    for g in range(_RSUB // _BL):
        row = ((_RSUB // _BL) * s + g) * 2 * _BL + c * _BL
        pltpu.sync_copy(
            loc_hbm.at[pl.ds(row, _BL)], lin_v.at[pl.ds(g * _BL, _BL)]
        )

    def _scat(k, carry):
        pltpu.sync_copy(ones_v, shared.at[lin_v.at[k]], add=True)
        return carry

    lax.fori_loop(0, _RSUB, _scat, 0)
    plsc.subcore_barrier()

    n_out = _HALF // _NS
    pltpu.sync_copy(
        shared.at[pl.ds(zbase, n_out)],
        mask_hbm.at[pl.ds(c * _HALF + zbase, n_out)],
    )


_sc_scatter = pl.kernel(
    _sc_body,
    out_type=jax.ShapeDtypeStruct((_NVOX,), jnp.float32),
    mesh=plsc.VectorSubcoreMesh(core_axis_name="c", subcore_axis_name="s"),
    scratch_types=[
        pltpu.VMEM_SHARED((_SPM,), jnp.float32),
        pltpu.VMEM((_RSUB, 128), jnp.int32),
        pltpu.VMEM((128,), jnp.float32),
        pltpu.VMEM((_ZB,), jnp.float32),
    ],
)


def _tc_body(mask_ref, feat_ref, out_ref):
    m = mask_ref[...]
    f = feat_ref[...]
    out_ref[...] = jnp.where(m[None, :, :, None] != 0.0, f, 0.0)


_tc_expand = pl.pallas_call(
    _tc_body,
    grid=(_D0,),
    in_specs=[
        pl.BlockSpec((_D1, _D2), lambda i: (i, 0)),
        pl.BlockSpec((1, 1, 1, _C), lambda i: (0, 0, 0, 0)),
    ],
    out_specs=pl.BlockSpec((1, _D1, _D2, _C), lambda i: (i, 0, 0, 0)),
    out_shape=jax.ShapeDtypeStruct((_D0, _D1, _D2, _C), jnp.float32),
    compiler_params=pltpu.CompilerParams(
        dimension_semantics=("arbitrary",),
    ),
)


@jax.jit
def kernel(voxel_features, indices):
    loc = _tc_linearize(indices.astype(jnp.int32))
    mask = _sc_scatter(loc)
    mask2 = mask.reshape(_D0 * _D1, _D2)
    feat = voxel_features.reshape(1, 1, 1, _C)
    return _tc_expand(mask2, feat)


# speedup vs baseline: 1.8737x; 1.8737x over previous
import jax
import jax.numpy as jnp
from jax import lax
from jax.experimental import pallas as pl
from jax.experimental.pallas import tpu as pltpu
from jax.experimental.pallas import tpu_sc as plsc

_D0, _D1, _D2, _C = 128, 128, 128, 32
_NVOX = _D0 * _D1 * _D2
_HALF = _NVOX // 2
_NC, _NS, _L = 2, 16, 16
_N = 200000
_PER_S = 12544
_KCH = _PER_S // 128
_HSTG = 2
_PER_STG = _PER_S // _HSTG
_KSTG = _KCH // _HSTG
_ZB = 2048
_SPM = _HALF + 16
_ZREP = _HALF // _NS // _ZB


def _sc_body(idx_hbm, mask_hbm, shared, x_v, y_v, z_v, lin_v, ones_v, zb_v):
    c = lax.axis_index("c")
    s = lax.axis_index("s")

    def _zset(i, carry):
        zb_v[pl.ds(i * _L, _L)] = jnp.zeros((_L,), jnp.float32)
        return carry

    lax.fori_loop(0, _ZB // _L, _zset, 0)
    for t in range(128 // _L):
        ones_v[pl.ds(t * _L, _L)] = jnp.ones((_L,), jnp.float32)
    zbase = s * (_HALF // _NS)
    for r in range(_ZREP):
        pltpu.sync_copy(zb_v, shared.at[pl.ds(zbase + r * _ZB, _ZB)])
    plsc.subcore_barrier()

    start = s * _PER_S - ((s + 1) >> 4) * (_PER_S * _NS - _N)
    lane = lax.iota(jnp.int32, _L)
    for h in range(_HSTG):
        base = start + h * _PER_STG
        pltpu.sync_copy(idx_hbm.at[pl.ds(base, _PER_STG)], x_v)
        pltpu.sync_copy(idx_hbm.at[pl.ds(_N + base, _PER_STG)], y_v)
        pltpu.sync_copy(idx_hbm.at[pl.ds(2 * _N + base, _PER_STG)], z_v)

        def _chunk(k, carry):
            for t in range(128 // _L):
                off = k * 128 + t * _L
                i0 = x_v[pl.ds(off, _L)]
                i1 = y_v[pl.ds(off, _L)]
                i2 = z_v[pl.ds(off, _L)]
                lin = i0 * (_D1 * _D2) + i1 * _D2 + i2
                loc = lin - c * _HALF
                inb = (loc >= 0) & (loc < _HALF)
                loc = jnp.where(inb, loc, _HALF + lane)
                lin_v[h * _KSTG + k, pl.ds(t * _L, _L)] = loc
            return carry

        lax.fori_loop(0, _KSTG, _chunk, 0)

    def _scat(k, carry):
        pltpu.sync_copy(ones_v, shared.at[lin_v.at[k]], add=True)
        return carry

    lax.fori_loop(0, _KCH, _scat, 0)
    plsc.subcore_barrier()

    n_out = _HALF // _NS
    pltpu.sync_copy(
        shared.at[pl.ds(zbase, n_out)],
        mask_hbm.at[pl.ds(c * _HALF + zbase, n_out)],
    )


_sc_scatter = pl.kernel(
    _sc_body,
    out_type=jax.ShapeDtypeStruct((_NVOX,), jnp.float32),
    mesh=plsc.VectorSubcoreMesh(core_axis_name="c", subcore_axis_name="s"),
    scratch_types=[
        pltpu.VMEM_SHARED((_SPM,), jnp.float32),
        pltpu.VMEM((_PER_STG,), jnp.int32),
        pltpu.VMEM((_PER_STG,), jnp.int32),
        pltpu.VMEM((_PER_STG,), jnp.int32),
        pltpu.VMEM((_KCH, 128), jnp.int32),
        pltpu.VMEM((128,), jnp.float32),
        pltpu.VMEM((_ZB,), jnp.float32),
    ],
)


def _tc_body(mask_ref, feat_ref, out_ref):
    m = mask_ref[...]
    f = feat_ref[...]
    out_ref[...] = jnp.where(m[:, :, None] != 0.0, f, 0.0)


_BROW = 256

_tc_expand = pl.pallas_call(
    _tc_body,
    grid=(_NVOX // 128 // _BROW,),
    in_specs=[
        pl.BlockSpec((_BROW, 128), lambda i: (i, 0)),
        pl.BlockSpec((1, 1, _C), lambda i: (0, 0, 0)),
    ],
    out_specs=pl.BlockSpec((_BROW, 128, _C), lambda i: (i, 0, 0)),
    out_shape=jax.ShapeDtypeStruct((_NVOX // 128, 128, _C), jnp.float32),
    compiler_params=pltpu.CompilerParams(
        dimension_semantics=("arbitrary",),
    ),
)


@jax.jit
def kernel(voxel_features, indices):
    idx_t = indices.astype(jnp.int32).T.reshape(3 * _N)
    mask = _sc_scatter(idx_t)
    mask2 = mask.reshape(_NVOX // 128, 128)
    feat = voxel_features.reshape(1, 1, _C)
    grid = _tc_expand(mask2, feat)
    return grid.reshape(_D0, _D1, _D2, _C)


# speedup vs baseline: 1.8781x; 1.0023x over previous
import jax
import jax.numpy as jnp
from jax import lax
from jax.experimental import pallas as pl
from jax.experimental.pallas import tpu as pltpu
from jax.experimental.pallas import tpu_sc as plsc

_D0, _D1, _D2, _C = 128, 128, 128, 32
_NVOX = _D0 * _D1 * _D2
_HALF = _NVOX // 2
_NC, _NS, _L = 2, 16, 16
_N = 200000
_PER_S = 12544
_KCH = _PER_S // 128
_HSTG = 2
_PER_STG = _PER_S // _HSTG
_KSTG = _KCH // _HSTG
_ZB = 4096
_SPM = _HALF + 16
_ZREP = _HALF // _NS // _ZB


def _sc_body(idx_hbm, mask_hbm, shared, x_v, y_v, z_v, lin_v, ones_v, zb_v):
    c = lax.axis_index("c")
    s = lax.axis_index("s")

    def _zset(i, carry):
        zb_v[pl.ds(i * _L, _L)] = jnp.zeros((_L,), jnp.float32)
        return carry

    lax.fori_loop(0, _ZB // _L, _zset, 0)
    for t in range(128 // _L):
        ones_v[pl.ds(t * _L, _L)] = jnp.ones((_L,), jnp.float32)
    zbase = s * (_HALF // _NS)
    for r in range(_ZREP):
        pltpu.sync_copy(zb_v, shared.at[pl.ds(zbase + r * _ZB, _ZB)])
    plsc.subcore_barrier()

    start = s * _PER_S - ((s + 1) >> 4) * (_PER_S * _NS - _N)
    lane = lax.iota(jnp.int32, _L)
    for h in range(_HSTG):
        base = start + h * _PER_STG
        pltpu.sync_copy(idx_hbm.at[pl.ds(base, _PER_STG)], x_v)
        pltpu.sync_copy(idx_hbm.at[pl.ds(_N + base, _PER_STG)], y_v)
        pltpu.sync_copy(idx_hbm.at[pl.ds(2 * _N + base, _PER_STG)], z_v)

        def _chunk(k, carry):
            for t in range(128 // _L):
                off = k * 128 + t * _L
                i0 = x_v[pl.ds(off, _L)]
                i1 = y_v[pl.ds(off, _L)]
                i2 = z_v[pl.ds(off, _L)]
                lin = i0 * (_D1 * _D2) + i1 * _D2 + i2
                loc = lin - c * _HALF
                inb = (loc >= 0) & (loc < _HALF)
                loc = jnp.where(inb, loc, _HALF + lane)
                lin_v[h * _KSTG + k, pl.ds(t * _L, _L)] = loc
            return carry

        lax.fori_loop(0, _KSTG, _chunk, 0)

    def _scat(k, carry):
        pltpu.sync_copy(ones_v, shared.at[lin_v.at[k]], add=True)
        return carry

    lax.fori_loop(0, _KCH, _scat, 0)
    plsc.subcore_barrier()

    n_out = _HALF // _NS
    pltpu.sync_copy(
        shared.at[pl.ds(zbase, n_out)],
        mask_hbm.at[pl.ds(c * _HALF + zbase, n_out)],
    )


_sc_scatter = pl.kernel(
    _sc_body,
    out_type=jax.ShapeDtypeStruct((_NVOX,), jnp.float32),
    mesh=plsc.VectorSubcoreMesh(core_axis_name="c", subcore_axis_name="s"),
    scratch_types=[
        pltpu.VMEM_SHARED((_SPM,), jnp.float32),
        pltpu.VMEM((_PER_STG,), jnp.int32),
        pltpu.VMEM((_PER_STG,), jnp.int32),
        pltpu.VMEM((_PER_STG,), jnp.int32),
        pltpu.VMEM((_KCH, 128), jnp.int32),
        pltpu.VMEM((128,), jnp.float32),
        pltpu.VMEM((_ZB,), jnp.float32),
    ],
)


def _tc_body(mask_ref, feat_ref, out_ref):
    m = mask_ref[...]
    f = feat_ref[...]
    out_ref[...] = jnp.where(m[:, :, None] != 0.0, f, 0.0)


_BROW = 256

_tc_expand = pl.pallas_call(
    _tc_body,
    grid=(_NVOX // 128 // _BROW,),
    in_specs=[
        pl.BlockSpec((_BROW, 128), lambda i: (i, 0)),
        pl.BlockSpec((1, 1, _C), lambda i: (0, 0, 0)),
    ],
    out_specs=pl.BlockSpec((_BROW, 128, _C), lambda i: (i, 0, 0)),
    out_shape=jax.ShapeDtypeStruct((_NVOX // 128, 128, _C), jnp.float32),
    compiler_params=pltpu.CompilerParams(
        dimension_semantics=("arbitrary",),
    ),
)


@jax.jit
def kernel(voxel_features, indices):
    idx_t = indices.astype(jnp.int32).T.reshape(3 * _N)
    mask = _sc_scatter(idx_t)
    mask2 = mask.reshape(_NVOX // 128, 128)
    feat = voxel_features.reshape(1, 1, _C)
    grid = _tc_expand(mask2, feat)
    return grid.reshape(_D0, _D1, _D2, _C)
